# Initial kernel scaffold; baseline (speedup 1.0000x reference)
import jax
import jax.numpy as jnp
from jax.experimental import pallas as pl


def kernel(x, norm1_scale, norm1_bias, conv1_w, conv1_b, norm2_scale, norm2_bias, conv2_w, conv2_b, cb0, cb1, cb2, cb3, attn0, attn1, attn2):
    raise NotImplementedError("write your pallas kernel here")


# speedup vs baseline: 1.5776x; 1.5776x over previous
import functools

import jax
import jax.numpy as jnp
from jax import lax
from jax.experimental import pallas as pl
from jax.experimental.pallas import tpu as pltpu
from jax.experimental.pallas import tpu_sc as plsc

_B, _FEAT, _ZCH, _H, _W = 32, 384, 256, 16, 16
_HW = _H * _W
_N = _B * _HW
_CB0 = 512
_HI = jax.lax.Precision.HIGHEST
_DEF = jax.lax.Precision.DEFAULT


def _argmin_first(scores):
  m = jnp.min(scores, axis=1, keepdims=True)
  io = jax.lax.broadcasted_iota(jnp.int32, scores.shape, 1)
  big = jnp.int32(scores.shape[1])
  idx = jnp.min(jnp.where(scores == m, io, big), axis=1)
  return idx, m


def _gsel(c, group_size):
  ng = c // group_size
  ci = lax.broadcasted_iota(jnp.int32, (c, ng), 0)
  gi = lax.broadcasted_iota(jnp.int32, (c, ng), 1)
  return (ci // group_size == gi).astype(jnp.float32)


def _group_norm_swish(x, group_size, scale_row, bias_row):
  t, c = x.shape
  sel = _gsel(c, group_size)
  cs = jnp.sum(x, axis=0, keepdims=True)
  cq = jnp.sum(x * x, axis=0, keepdims=True)
  cnt = jnp.float32(t * group_size)
  mu = lax.dot_general(cs, sel, (((1,), (0,)), ((), ())), precision=_HI) / cnt
  ex2 = lax.dot_general(cq, sel, (((1,), (0,)), ((), ())), precision=_HI) / cnt
  var = ex2 - mu * mu
  rstd = lax.rsqrt(var + 1e-6)
  mu_c = lax.dot_general(mu, sel, (((1,), (1,)), ((), ())), precision=_HI)
  rstd_c = lax.dot_general(rstd, sel, (((1,), (1,)), ((), ())), precision=_HI)
  a = scale_row * rstd_c
  b = bias_row - mu_c * a
  xn = x * a + b
  return xn / (1.0 + jnp.exp(-xn))


def _conv3x3(xpad_ref, w_ref, bias_row):
  accs = []
  for dj in (-1, 0, 1):
    acc = None
    for di in (-1, 0, 1):
      k = (di + 1) * 3 + (dj + 1)
      xs = xpad_ref[16 + 16 * di:272 + 16 * di, :]
      p = lax.dot_general(xs, w_ref[k], (((1,), (0,)), ((), ())),
                          precision=_DEF)
      acc = p if acc is None else acc + p
    accs.append(acc)
  jcol = lax.broadcasted_iota(jnp.int32, (_HW, 1), 0) % _W
  h = accs[1]
  h = h + jnp.where(jcol != 0, 1.0, 0.0) * pltpu.roll(accs[0], 1, 0)
  h = h + jnp.where(jcol != _W - 1, 1.0, 0.0) * pltpu.roll(accs[2], _HW - 1, 0)
  return h + bias_row


def _tables_body(cb0_r, cb1_r, cb2_r, cb3_r, a0_r, a1_r, a2_r,
                 t1_o, t2_o, t3_o, gcol_o, n0_o, tabs_o):
  c0 = cb0_r[...]
  c1 = cb1_r[...]
  c2 = cb2_r[...]
  c3 = cb3_r[...]
  n0 = jnp.sum(c0 * c0, axis=1, keepdims=True)
  n1 = jnp.sum(c1 * c1, axis=1, keepdims=True)
  n2 = jnp.sum(c2 * c2, axis=1, keepdims=True)
  n3 = jnp.sum(c3 * c3, axis=1, keepdims=True)

  def pdist(na, a, nb, b):
    return ((na + nb.T)
            - 2.0 * lax.dot_general(a, b, (((1,), (1,)), ((), ())),
                                    precision=_DEF))

  def attn_t(attn, cb):
    return lax.dot_general(attn, cb, (((0,), (0,)), ((), ())), precision=_DEF)

  d01 = pdist(n0, c0, n1, c1)
  next1, min01 = _argmin_first(d01)
  cba0 = attn_t(a0_r[...], c0)
  al1 = jnp.sum((c1 - cba0) ** 2, axis=1, keepdims=True)

  d12 = pdist(n1, c1, n2, c2)
  next2, min12 = _argmin_first(d12)
  cba1 = attn_t(a1_r[...], c1)
  al2 = jnp.sum((c2 - cba1) ** 2, axis=1, keepdims=True)

  d23 = pdist(n2, c2, n3, c3)
  next3, min23 = _argmin_first(d23)
  cba2 = attn_t(a2_r[...], c2)
  al3 = jnp.sum((c3 - cba2) ** 2, axis=1, keepdims=True)

  def onehot(idx_col, n):
    io = lax.broadcasted_iota(jnp.int32, (idx_col.shape[0], n), 1)
    return (io == idx_col).astype(jnp.float32)

  def sel(oh, col):
    return lax.dot_general(oh, col, (((1,), (0,)), ((), ())), precision=_HI)

  t1i = next1[:, None]
  oh1 = onehot(t1i, 256)
  t2i = sel(oh1, next2[:, None].astype(jnp.float32)).astype(jnp.int32)
  oh2 = onehot(t2i, 128)
  t3i = sel(oh2, next3[:, None].astype(jnp.float32)).astype(jnp.int32)
  oh3 = onehot(t3i, 64)

  t1_o[...] = sel(oh1, c1)
  t2_o[...] = sel(oh2, c2)
  t3_o[...] = sel(oh3, c3)
  g = (2.0 * min01
       + sel(oh1, al1 + 2.0 * min12)
       + sel(oh2, al2 + 2.0 * min23)
       + sel(oh3, al3))
  gcol_o[...] = g
  n0_o[...] = n0
  tabs_o[...] = jnp.concatenate(
      [g.T, t1i.astype(jnp.float32).T, t2i.astype(jnp.float32).T,
       t3i.astype(jnp.float32).T,
       jnp.zeros((4, _CB0), jnp.float32)], axis=0)


def _tables(cb0, cb1, cb2, cb3, attn0, attn1, attn2):
  out_shapes = (
      jax.ShapeDtypeStruct((_CB0, _ZCH), jnp.float32),
      jax.ShapeDtypeStruct((_CB0, _ZCH), jnp.float32),
      jax.ShapeDtypeStruct((_CB0, _ZCH), jnp.float32),
      jax.ShapeDtypeStruct((_CB0, 1), jnp.float32),
      jax.ShapeDtypeStruct((_CB0, 1), jnp.float32),
      jax.ShapeDtypeStruct((8, _CB0), jnp.float32),
  )
  return pl.pallas_call(_tables_body, out_shape=out_shapes)(
      cb0, cb1, cb2, cb3, attn0, attn1, attn2)


def _main_body(xt_r, w1_r, w2_r, ns1_r, nb1_r, ns2_r, nb2_r, c1b_r, c2b_r,
               cb0t_r, cb0n2_r, tabs_r, idx_o, loss_o, xpad, hpad):
  b = pl.program_id(0)
  x = xt_r[0]
  xs = _group_norm_swish(x, _FEAT // 32, ns1_r[...], nb1_r[...])
  xpad[0:16, :] = jnp.zeros((16, _FEAT), jnp.float32)
  xpad[272:288, :] = jnp.zeros((16, _FEAT), jnp.float32)
  xpad[16:272, :] = xs
  h = _conv3x3(xpad, w1_r, c1b_r[...])
  hs = _group_norm_swish(h, _ZCH // 32, ns2_r[...], nb2_r[...])
  hpad[0:16, :] = jnp.zeros((16, _ZCH), jnp.float32)
  hpad[272:288, :] = jnp.zeros((16, _ZCH), jnp.float32)
  hpad[16:272, :] = hs
  z = _conv3x3(hpad, w2_r, c2b_r[...])

  zz = jnp.sum(z * z, axis=1, keepdims=True)
  scores = (zz + cb0n2_r[...]) - 2.0 * lax.dot_general(
      z, cb0t_r[...], (((1,), (0,)), ((), ())), precision=_DEF)
  idx, dmin = _argmin_first(scores)

  io = lax.broadcasted_iota(jnp.int32, (_HW, _CB0), 1)
  oh = (io == idx[:, None]).astype(jnp.float32)
  tabs = tabs_r[...]
  gsum = jnp.sum(oh * tabs[0:1, :])
  lsum = 2.0 * jnp.sum(dmin) + gsum
  idx1 = jnp.sum(oh * tabs[1:2, :], axis=1).astype(jnp.int32)
  idx2 = jnp.sum(oh * tabs[2:3, :], axis=1).astype(jnp.int32)
  idx3 = jnp.sum(oh * tabs[3:4, :], axis=1).astype(jnp.int32)
  idx_o[0] = jnp.concatenate(
      [idx.reshape(1, _HW), idx1.reshape(1, _HW), idx2.reshape(1, _HW),
       idx3.reshape(1, _HW), jnp.zeros((4, _HW), jnp.int32)], axis=0)

  @pl.when(b == 0)
  def _():
    loss_o[0, 0] = lsum

  @pl.when(b != 0)
  def _():
    loss_o[0, 0] += lsum


def _main(xt, w1r, w2r, ns1, nb1, ns2, nb2, c1b, c2b, cb0t, cb0n2, tabs):
  grid = (_B,)
  in_specs = [
      pl.BlockSpec((1, _HW, _FEAT), lambda b: (b, 0, 0)),
      pl.BlockSpec((9, _FEAT, _ZCH), lambda b: (0, 0, 0)),
      pl.BlockSpec((9, _ZCH, _ZCH), lambda b: (0, 0, 0)),
      pl.BlockSpec((1, _FEAT), lambda b: (0, 0)),
      pl.BlockSpec((1, _FEAT), lambda b: (0, 0)),
      pl.BlockSpec((1, _ZCH), lambda b: (0, 0)),
      pl.BlockSpec((1, _ZCH), lambda b: (0, 0)),
      pl.BlockSpec((1, _ZCH), lambda b: (0, 0)),
      pl.BlockSpec((1, _ZCH), lambda b: (0, 0)),
      pl.BlockSpec((_ZCH, _CB0), lambda b: (0, 0)),
      pl.BlockSpec((1, _CB0), lambda b: (0, 0)),
      pl.BlockSpec((8, _CB0), lambda b: (0, 0)),
  ]
  out_specs = [
      pl.BlockSpec((1, 8, _HW), lambda b: (b, 0, 0)),
      pl.BlockSpec(memory_space=pltpu.SMEM),
  ]
  out_shapes = [
      jax.ShapeDtypeStruct((_B, 8, _HW), jnp.int32),
      jax.ShapeDtypeStruct((1, 1), jnp.float32),
  ]
  scratch = [
      pltpu.VMEM((288, _FEAT), jnp.float32),
      pltpu.VMEM((288, _ZCH), jnp.float32),
  ]
  return pl.pallas_call(
      _main_body, grid=grid, in_specs=in_specs, out_specs=out_specs,
      out_shape=out_shapes, scratch_shapes=scratch)(
          xt, w1r, w2r, ns1, nb1, ns2, nb2, c1b, c2b, cb0t, cb0n2, tabs)


_NC, _NS = 2, 16
_NW = _NC * _NS
_CHUNK = _N // _NW


def _sc_gather_body(t0, t1, t2, t3, idx_hbm, o0, o1, o2, o3,
                    idx_v, rows_v, sem):
  wid = lax.axis_index("s") * _NC + lax.axis_index("c")
  base = wid * _CHUNK
  pltpu.sync_copy(idx_hbm.at[pl.ds(base, _CHUNK)], idx_v)
  for t_hbm, o_hbm in ((t0, o0), (t1, o1), (t2, o2), (t3, o3)):
    pltpu.async_copy(t_hbm.at[idx_v], rows_v, sem).wait()
    pltpu.sync_copy(rows_v, o_hbm.at[pl.ds(base, _CHUNK)])


@functools.cache
def _sc_gather():
  return pl.kernel(
      _sc_gather_body,
      mesh=plsc.VectorSubcoreMesh(core_axis_name="c", subcore_axis_name="s",
                                  num_cores=_NC, num_subcores=_NS),
      out_type=[jax.ShapeDtypeStruct((_N, _ZCH), jnp.float32)] * 4,
      scratch_types=[
          pltpu.VMEM((_CHUNK,), jnp.int32),
          pltpu.VMEM((_CHUNK, _ZCH), jnp.float32),
          pltpu.SemaphoreType.DMA,
      ],
  )


def kernel(x, norm1_scale, norm1_bias, conv1_w, conv1_b, norm2_scale,
           norm2_bias, conv2_w, conv2_b, cb0, cb1, cb2, cb3,
           attn0, attn1, attn2):
  xt = jnp.transpose(x.reshape(_B, _FEAT, _HW), (0, 2, 1))
  w1r = jnp.transpose(conv1_w, (2, 3, 1, 0)).reshape(9, _FEAT, _ZCH)
  w2r = jnp.transpose(conv2_w, (2, 3, 1, 0)).reshape(9, _ZCH, _ZCH)

  t1t, t2t, t3t, _gcol, n0c, tabs = _tables(cb0, cb1, cb2, cb3,
                                            attn0, attn1, attn2)
  idx_all, loss_pre = _main(
      xt, w1r, w2r,
      norm1_scale.reshape(1, _FEAT), norm1_bias.reshape(1, _FEAT),
      norm2_scale.reshape(1, _ZCH), norm2_bias.reshape(1, _ZCH),
      conv1_b.reshape(1, _ZCH), conv2_b.reshape(1, _ZCH),
      cb0.T, n0c.reshape(1, _CB0), tabs)

  idx0 = idx_all[:, 0, :].reshape(_N)
  outs = _sc_gather()(cb0, t1t, t2t, t3t, idx0)
  zqs = [jnp.transpose(o.reshape(_B, _H, _W, _ZCH), (0, 3, 1, 2))
         for o in outs]
  loss = loss_pre[0, 0] * jnp.float32(1.0 / (_N * _ZCH))
  inds = jnp.transpose(idx_all[:, 0:4, :], (1, 0, 2)).reshape(4, _B, _H, _W)
  return (zqs[0], zqs[1], zqs[2], zqs[3], loss, inds)
